# Initial kernel scaffold; baseline (speedup 1.0000x reference)
import jax
import jax.numpy as jnp
from jax.experimental import pallas as pl


def kernel(x, edges, edge_index, Wq, Wk, Wv, Wek, Wev, Wexp, Wout):
    raise NotImplementedError("write your pallas kernel here")


# speedup vs baseline: 8.3953x; 8.3953x over previous
import functools

import jax
import jax.numpy as jnp
from jax.experimental import pallas as pl
from jax.experimental.pallas import tpu as pltpu

N = 10000
E = 320000
DIM = 128
HEADS = 8
HEAD_DIM = DIM // HEADS
EXP_HEADS = 8


def _proj3_body(x_ref, wq_ref, wk_ref, wv_ref, q_ref, k_ref, v_ref, *, scale):
    x = x_ref[...]
    q_ref[...] = jnp.dot(x, wq_ref[...].T, preferred_element_type=jnp.float32)
    k_ref[...] = jnp.dot(x, wk_ref[...].T, preferred_element_type=jnp.float32) * scale
    v_ref[...] = jnp.dot(x, wv_ref[...].T, preferred_element_type=jnp.float32)


def _edgeproj_body(e_ref, wek_ref, wev_ref, ek_ref, ev_ref, *, scale):
    e = e_ref[...]
    ek_ref[...] = jnp.dot(e, wek_ref[...].T, preferred_element_type=jnp.float32) * scale
    ev_ref[...] = jnp.dot(e, wev_ref[...].T, preferred_element_type=jnp.float32)


def _finish_body(v_ref, num_ref, den_ref, wout_ref, o_ref):
    v = v_ref[...]
    den = den_ref[...]
    agg = num_ref[...].reshape(-1, EXP_HEADS, HEAD_DIM) / (den + 1e-9)[:, :, None]
    out = v - agg.reshape(-1, EXP_HEADS * HEAD_DIM)
    o_ref[...] = jnp.dot(out, wout_ref[...].T, preferred_element_type=jnp.float32)


def kernel(x, edges, edge_index, Wq, Wk, Wv, Wek, Wev, Wexp, Wout):
    scale = HEAD_DIM ** -0.5
    BN = 1000
    q, k, v = pl.pallas_call(
        functools.partial(_proj3_body, scale=scale),
        grid=(N // BN,),
        in_specs=[
            pl.BlockSpec((BN, DIM), lambda i: (i, 0)),
            pl.BlockSpec((DIM, DIM), lambda i: (0, 0)),
            pl.BlockSpec((DIM, DIM), lambda i: (0, 0)),
            pl.BlockSpec((DIM, DIM), lambda i: (0, 0)),
        ],
        out_specs=[
            pl.BlockSpec((BN, DIM), lambda i: (i, 0)),
            pl.BlockSpec((BN, DIM), lambda i: (i, 0)),
            pl.BlockSpec((BN, DIM), lambda i: (i, 0)),
        ],
        out_shape=[jax.ShapeDtypeStruct((N, DIM), jnp.float32)] * 3,
    )(x, Wq, Wk, Wv)

    BE = 2000
    ek, ev = pl.pallas_call(
        functools.partial(_edgeproj_body, scale=scale),
        grid=(E // BE,),
        in_specs=[
            pl.BlockSpec((BE, DIM), lambda i: (i, 0)),
            pl.BlockSpec((DIM, DIM), lambda i: (0, 0)),
            pl.BlockSpec((DIM, DIM), lambda i: (0, 0)),
        ],
        out_specs=[
            pl.BlockSpec((BE, DIM), lambda i: (i, 0)),
            pl.BlockSpec((BE, DIM), lambda i: (i, 0)),
        ],
        out_shape=[jax.ShapeDtypeStruct((E, DIM), jnp.float32)] * 2,
    )(edges, Wek, Wev)

    row = edge_index[0]
    col = edge_index[1]
    qe = jnp.take(q, row, axis=0).reshape(E, HEADS, HEAD_DIM)
    ke = jnp.take(k, col, axis=0).reshape(E, HEADS, HEAD_DIM) + ek.reshape(E, HEADS, HEAD_DIM)
    scores = jnp.sum(qe * ke, axis=-1) @ Wexp.T
    smax = jax.ops.segment_max(scores, row, num_segments=N)
    smax = jnp.where(jnp.isfinite(smax), smax, 0.0)
    ex = jnp.exp(scores - jnp.take(smax, row, axis=0))
    den = jax.ops.segment_sum(ex, row, num_segments=N)
    ve = jnp.take(v, col, axis=0).reshape(E, EXP_HEADS, HEAD_DIM) + ev.reshape(E, EXP_HEADS, HEAD_DIM)
    msg = ex[:, :, None] * ve
    num = jax.ops.segment_sum(msg.reshape(E, DIM), row, num_segments=N)

    out = pl.pallas_call(
        _finish_body,
        grid=(N // BN,),
        in_specs=[
            pl.BlockSpec((BN, DIM), lambda i: (i, 0)),
            pl.BlockSpec((BN, DIM), lambda i: (i, 0)),
            pl.BlockSpec((BN, EXP_HEADS), lambda i: (i, 0)),
            pl.BlockSpec((DIM, DIM), lambda i: (0, 0)),
        ],
        out_specs=pl.BlockSpec((BN, DIM), lambda i: (i, 0)),
        out_shape=jax.ShapeDtypeStruct((N, DIM), jnp.float32),
    )(v, num, den, Wout)
    return out


# speedup vs baseline: 14.0413x; 1.6725x over previous
import functools

import jax
import jax.numpy as jnp
from jax.experimental import pallas as pl
from jax.experimental.pallas import tpu as pltpu

N = 10000
E = 320000
DIM = 128
HEADS = 8
HEAD_DIM = DIM // HEADS
EXP_HEADS = 8
CMB = DIM + EXP_HEADS


def _proj_body(x_ref, wq_ref, wk_ref, wv_ref, q_ref, kv_ref, *, scale):
    x = x_ref[...]
    q_ref[...] = jnp.dot(x, wq_ref[...].T, preferred_element_type=jnp.float32)
    kv_ref[:, :DIM] = jnp.dot(x, wk_ref[...].T, preferred_element_type=jnp.float32) * scale
    kv_ref[:, DIM:] = jnp.dot(x, wv_ref[...].T, preferred_element_type=jnp.float32)


def _edge_body(e_ref, qe_ref, kve_ref, wek_ref, wev_ref, p_ref, r_ref, cmb_ref, *, scale):
    e = e_ref[...]
    ek = jnp.dot(e, wek_ref[...].T, preferred_element_type=jnp.float32) * scale
    ev = jnp.dot(e, wev_ref[...].T, preferred_element_type=jnp.float32)
    qe = qe_ref[...]
    ke = kve_ref[:, :DIM] + ek
    scores = jnp.dot(qe * ke, p_ref[...], preferred_element_type=jnp.float32)
    ex = jnp.exp(jnp.minimum(scores, 75.0))
    ex128 = jnp.dot(ex, r_ref[...], preferred_element_type=jnp.float32)
    ve = kve_ref[:, DIM:] + ev
    cmb_ref[:, :DIM] = ex128 * ve
    cmb_ref[:, DIM:] = ex


def _finish_body(v_ref, nd_ref, wout_ref, o_ref):
    v = v_ref[...]
    den = nd_ref[:, DIM:]
    agg = nd_ref[:, :DIM].reshape(-1, EXP_HEADS, HEAD_DIM) / (den + 1e-9)[:, :, None]
    out = v - agg.reshape(-1, DIM)
    o_ref[...] = jnp.dot(out, wout_ref[...].T, preferred_element_type=jnp.float32)


def kernel(x, edges, edge_index, Wq, Wk, Wv, Wek, Wev, Wexp, Wout):
    scale = HEAD_DIM ** -0.5
    BN = 1000
    q, kv = pl.pallas_call(
        functools.partial(_proj_body, scale=scale),
        grid=(N // BN,),
        in_specs=[
            pl.BlockSpec((BN, DIM), lambda i: (i, 0)),
            pl.BlockSpec((DIM, DIM), lambda i: (0, 0)),
            pl.BlockSpec((DIM, DIM), lambda i: (0, 0)),
            pl.BlockSpec((DIM, DIM), lambda i: (0, 0)),
        ],
        out_specs=[
            pl.BlockSpec((BN, DIM), lambda i: (i, 0)),
            pl.BlockSpec((BN, 2 * DIM), lambda i: (i, 0)),
        ],
        out_shape=[
            jax.ShapeDtypeStruct((N, DIM), jnp.float32),
            jax.ShapeDtypeStruct((N, 2 * DIM), jnp.float32),
        ],
    )(x, Wq, Wk, Wv)

    row = edge_index[0]
    col = edge_index[1]
    qe = jnp.take(q, row, axis=0)
    kve = jnp.take(kv, col, axis=0)

    P = jnp.repeat(Wexp.T, HEAD_DIM, axis=0)
    R = jnp.repeat(jnp.eye(EXP_HEADS, dtype=jnp.float32), HEAD_DIM, axis=1)

    BE = 2000
    cmb = pl.pallas_call(
        functools.partial(_edge_body, scale=scale),
        grid=(E // BE,),
        in_specs=[
            pl.BlockSpec((BE, DIM), lambda i: (i, 0)),
            pl.BlockSpec((BE, DIM), lambda i: (i, 0)),
            pl.BlockSpec((BE, 2 * DIM), lambda i: (i, 0)),
            pl.BlockSpec((DIM, DIM), lambda i: (0, 0)),
            pl.BlockSpec((DIM, DIM), lambda i: (0, 0)),
            pl.BlockSpec((DIM, EXP_HEADS), lambda i: (0, 0)),
            pl.BlockSpec((EXP_HEADS, DIM), lambda i: (0, 0)),
        ],
        out_specs=pl.BlockSpec((BE, CMB), lambda i: (i, 0)),
        out_shape=jax.ShapeDtypeStruct((E, CMB), jnp.float32),
    )(edges, qe, kve, Wek, Wev, P, R)

    num_den = jax.ops.segment_sum(cmb, row, num_segments=N)

    out = pl.pallas_call(
        _finish_body,
        grid=(N // BN,),
        in_specs=[
            pl.BlockSpec((BN, DIM), lambda i: (i, 0)),
            pl.BlockSpec((BN, CMB), lambda i: (i, 0)),
            pl.BlockSpec((DIM, DIM), lambda i: (0, 0)),
        ],
        out_specs=pl.BlockSpec((BN, DIM), lambda i: (i, 0)),
        out_shape=jax.ShapeDtypeStruct((N, DIM), jnp.float32),
    )(kv[:, DIM:], num_den, Wout)
    return out


# speedup vs baseline: 18.8234x; 1.3406x over previous
import functools

import jax
import jax.numpy as jnp
from jax import lax
from jax.experimental import pallas as pl
from jax.experimental.pallas import tpu as pltpu
from jax.experimental.pallas import tpu_sc as plsc

N = 10000
E = 320000
DIM = 128
HEADS = 8
HEAD_DIM = DIM // HEADS
EXP_HEADS = 8

NGRP = E // 128
GRP_PER_SC = NGRP // 2
GRP_BASE = GRP_PER_SC // 16
GRP_REM = GRP_PER_SC % 16
ROWS_PER_TILE = 632
NP = 16 * ROWS_PER_TILE


def _proj_body(x_ref, wq_ref, wk_ref, wv_ref, q_ref, kv_ref, *, scale):
    x = x_ref[...]
    q_ref[...] = jnp.dot(x, wq_ref[...].T, preferred_element_type=jnp.float32)
    kv_ref[:, :DIM] = jnp.dot(x, wk_ref[...].T, preferred_element_type=jnp.float32) * scale
    kv_ref[:, DIM:] = jnp.dot(x, wv_ref[...].T, preferred_element_type=jnp.float32)


def _edge_body(e_ref, qe_ref, kve_ref, wek_ref, wev_ref, p_ref, r_ref,
               msg_ref, exb_ref, *, scale):
    e = e_ref[...]
    ek = jnp.dot(e, wek_ref[...].T, preferred_element_type=jnp.float32) * scale
    ev = jnp.dot(e, wev_ref[...].T, preferred_element_type=jnp.float32)
    qe = qe_ref[...]
    ke = kve_ref[:, :DIM] + ek
    scores = jnp.dot(qe * ke, p_ref[...], preferred_element_type=jnp.float32)
    ex = jnp.exp(jnp.minimum(scores, 75.0))
    ex128 = jnp.dot(ex, r_ref[...], preferred_element_type=jnp.float32)
    ve = kve_ref[:, DIM:] + ev
    msg_ref[...] = ex128 * ve
    exb_ref[...] = ex128


def _scatter_body(msg_hbm, exb_hbm, row_hbm, zro_hbm,
                  num_out, den_out,
                  idx2_v, stage_v, acc):
    c = lax.axis_index("c")
    s = lax.axis_index("s")
    base = c * GRP_PER_SC + s * GRP_BASE + jnp.minimum(s, GRP_REM)
    count = GRP_BASE + (s < GRP_REM).astype(jnp.int32)
    rslice = pl.ds(s * ROWS_PER_TILE, ROWS_PER_TILE)

    def phase(data_hbm, out_hbm):
        pltpu.sync_copy(zro_hbm.at[rslice], acc.at[rslice])
        plsc.subcore_barrier()

        def group_body(i, carry):
            g = base + i
            pltpu.sync_copy(row_hbm.at[pl.ds(g, 1)], idx2_v)
            pltpu.sync_copy(data_hbm.at[pl.ds(g * 128, 128)], stage_v)
            pltpu.sync_copy(stage_v, acc.at[idx2_v.at[0]], add=True)
            return carry

        lax.fori_loop(0, count, group_body, 0)
        plsc.subcore_barrier()
        pltpu.sync_copy(acc.at[rslice], out_hbm.at[c, rslice])
        plsc.subcore_barrier()

    phase(msg_hbm, num_out)
    phase(exb_hbm, den_out)


def _finish_body(v_ref, num_ref, den_ref, wout_ref, o_ref):
    v = v_ref[...]
    num = num_ref[0] + num_ref[1]
    den = den_ref[0] + den_ref[1]
    out = v - num / (den + 1e-9)
    o_ref[...] = jnp.dot(out, wout_ref[...].T, preferred_element_type=jnp.float32)


def kernel(x, edges, edge_index, Wq, Wk, Wv, Wek, Wev, Wexp, Wout):
    scale = HEAD_DIM ** -0.5
    BN = 1000
    q, kv = pl.pallas_call(
        functools.partial(_proj_body, scale=scale),
        grid=(N // BN,),
        in_specs=[
            pl.BlockSpec((BN, DIM), lambda i: (i, 0)),
            pl.BlockSpec((DIM, DIM), lambda i: (0, 0)),
            pl.BlockSpec((DIM, DIM), lambda i: (0, 0)),
            pl.BlockSpec((DIM, DIM), lambda i: (0, 0)),
        ],
        out_specs=[
            pl.BlockSpec((BN, DIM), lambda i: (i, 0)),
            pl.BlockSpec((BN, 2 * DIM), lambda i: (i, 0)),
        ],
        out_shape=[
            jax.ShapeDtypeStruct((N, DIM), jnp.float32),
            jax.ShapeDtypeStruct((N, 2 * DIM), jnp.float32),
        ],
    )(x, Wq, Wk, Wv)

    row = edge_index[0]
    col = edge_index[1]
    qe = jnp.take(q, row, axis=0)
    kve = jnp.take(kv, col, axis=0)

    P = jnp.repeat(Wexp.T, HEAD_DIM, axis=0)
    R = jnp.repeat(jnp.eye(EXP_HEADS, dtype=jnp.float32), HEAD_DIM, axis=1)

    BE = 2000
    msg, exb = pl.pallas_call(
        functools.partial(_edge_body, scale=scale),
        grid=(E // BE,),
        in_specs=[
            pl.BlockSpec((BE, DIM), lambda i: (i, 0)),
            pl.BlockSpec((BE, DIM), lambda i: (i, 0)),
            pl.BlockSpec((BE, 2 * DIM), lambda i: (i, 0)),
            pl.BlockSpec((DIM, DIM), lambda i: (0, 0)),
            pl.BlockSpec((DIM, DIM), lambda i: (0, 0)),
            pl.BlockSpec((DIM, EXP_HEADS), lambda i: (0, 0)),
            pl.BlockSpec((EXP_HEADS, DIM), lambda i: (0, 0)),
        ],
        out_specs=[
            pl.BlockSpec((BE, DIM), lambda i: (i, 0)),
            pl.BlockSpec((BE, DIM), lambda i: (i, 0)),
        ],
        out_shape=[
            jax.ShapeDtypeStruct((E, DIM), jnp.float32),
            jax.ShapeDtypeStruct((E, DIM), jnp.float32),
        ],
    )(edges, qe, kve, Wek, Wev, P, R)

    row2d = row.reshape(NGRP, 128)
    zros = jnp.zeros((NP, DIM), jnp.float32)
    num2, den2 = pl.kernel(
        _scatter_body,
        out_type=[
            jax.ShapeDtypeStruct((2, NP, DIM), jnp.float32),
            jax.ShapeDtypeStruct((2, NP, DIM), jnp.float32),
        ],
        mesh=plsc.VectorSubcoreMesh(core_axis_name="c", subcore_axis_name="s"),
        scratch_types=[
            pltpu.VMEM((1, 128), jnp.int32),
            pltpu.VMEM((128, DIM), jnp.float32),
            pltpu.VMEM_SHARED((NP, DIM), jnp.float32),
        ],
    )(msg, exb, row2d, zros)

    out = pl.pallas_call(
        _finish_body,
        grid=(N // BN,),
        in_specs=[
            pl.BlockSpec((BN, DIM), lambda i: (i, 0)),
            pl.BlockSpec((2, BN, DIM), lambda i: (0, i, 0)),
            pl.BlockSpec((2, BN, DIM), lambda i: (0, i, 0)),
            pl.BlockSpec((DIM, DIM), lambda i: (0, 0)),
        ],
        out_specs=pl.BlockSpec((BN, DIM), lambda i: (i, 0)),
        out_shape=jax.ShapeDtypeStruct((N, DIM), jnp.float32),
    )(kv[:, DIM:], num2, den2, Wout)
    return out


# speedup vs baseline: 19.0311x; 1.0110x over previous
import functools

import jax
import jax.numpy as jnp
from jax import lax
from jax.experimental import pallas as pl
from jax.experimental.pallas import tpu as pltpu
from jax.experimental.pallas import tpu_sc as plsc

N = 10000
E = 320000
DIM = 128
HEADS = 8
HEAD_DIM = DIM // HEADS
EXP_HEADS = 8

NGRP = E // 128
GRP_PER_SC = NGRP // 2
GRP_BASE = GRP_PER_SC // 16
GRP_REM = GRP_PER_SC % 16
ROWS_PER_TILE = 632
NP = 16 * ROWS_PER_TILE


def _proj_body(x_ref, wq_ref, wk_ref, wv_ref, q_ref, kv_ref, v_ref, *, scale):
    x = x_ref[...]
    q_ref[...] = jnp.dot(x, wq_ref[...].T, preferred_element_type=jnp.float32).astype(jnp.bfloat16)
    kv_ref[:, :DIM] = (jnp.dot(x, wk_ref[...].T, preferred_element_type=jnp.float32) * scale).astype(jnp.bfloat16)
    v = jnp.dot(x, wv_ref[...].T, preferred_element_type=jnp.float32)
    kv_ref[:, DIM:] = v.astype(jnp.bfloat16)
    v_ref[...] = v


def _edge_body(e_ref, qe_ref, kve_ref, wek_ref, wev_ref, p_ref, r_ref,
               msg_ref, exb_ref, *, scale):
    e = e_ref[...]
    ek = jnp.dot(e, wek_ref[...].T, preferred_element_type=jnp.float32) * scale
    ev = jnp.dot(e, wev_ref[...].T, preferred_element_type=jnp.float32)
    qe = qe_ref[...].astype(jnp.float32)
    kve = kve_ref[...].astype(jnp.float32)
    ke = kve[:, :DIM] + ek
    scores = jnp.dot(qe * ke, p_ref[...], preferred_element_type=jnp.float32)
    ex = jnp.exp(jnp.minimum(scores, 75.0))
    ex128 = jnp.dot(ex, r_ref[...], preferred_element_type=jnp.float32)
    ve = kve[:, DIM:] + ev
    msg_ref[...] = ex128 * ve
    exb_ref[...] = ex128


def _scatter_body(msg_hbm, exb_hbm, row_hbm, zro_hbm,
                  num_out, den_out,
                  idx2_v, stage_v, acc):
    c = lax.axis_index("c")
    s = lax.axis_index("s")
    base = c * GRP_PER_SC + s * GRP_BASE + jnp.minimum(s, GRP_REM)
    count = GRP_BASE + (s < GRP_REM).astype(jnp.int32)
    rslice = pl.ds(s * ROWS_PER_TILE, ROWS_PER_TILE)

    def phase(data_hbm, out_hbm):
        pltpu.sync_copy(zro_hbm.at[rslice], acc.at[rslice])
        plsc.subcore_barrier()

        def group_body(i, carry):
            g = base + i
            pltpu.sync_copy(row_hbm.at[pl.ds(g, 1)], idx2_v)
            pltpu.sync_copy(data_hbm.at[pl.ds(g * 128, 128)], stage_v)
            pltpu.sync_copy(stage_v, acc.at[idx2_v.at[0]], add=True)
            return carry

        lax.fori_loop(0, count, group_body, 0)
        plsc.subcore_barrier()
        pltpu.sync_copy(acc.at[rslice], out_hbm.at[c, rslice])
        plsc.subcore_barrier()

    phase(msg_hbm, num_out)
    phase(exb_hbm, den_out)


def _finish_body(v_ref, num_ref, den_ref, wout_ref, o_ref):
    v = v_ref[...]
    num = num_ref[0] + num_ref[1]
    den = den_ref[0] + den_ref[1]
    out = v - num / (den + 1e-9)
    o_ref[...] = jnp.dot(out, wout_ref[...].T, preferred_element_type=jnp.float32)


def kernel(x, edges, edge_index, Wq, Wk, Wv, Wek, Wev, Wexp, Wout):
    scale = HEAD_DIM ** -0.5
    BN = 1000
    q, kv, vfull = pl.pallas_call(
        functools.partial(_proj_body, scale=scale),
        grid=(N // BN,),
        in_specs=[
            pl.BlockSpec((BN, DIM), lambda i: (i, 0)),
            pl.BlockSpec((DIM, DIM), lambda i: (0, 0)),
            pl.BlockSpec((DIM, DIM), lambda i: (0, 0)),
            pl.BlockSpec((DIM, DIM), lambda i: (0, 0)),
        ],
        out_specs=[
            pl.BlockSpec((BN, DIM), lambda i: (i, 0)),
            pl.BlockSpec((BN, 2 * DIM), lambda i: (i, 0)),
            pl.BlockSpec((BN, DIM), lambda i: (i, 0)),
        ],
        out_shape=[
            jax.ShapeDtypeStruct((N, DIM), jnp.bfloat16),
            jax.ShapeDtypeStruct((N, 2 * DIM), jnp.bfloat16),
            jax.ShapeDtypeStruct((N, DIM), jnp.float32),
        ],
    )(x, Wq, Wk, Wv)

    row = edge_index[0]
    col = edge_index[1]
    qe = jnp.take(q, row, axis=0)
    kve = jnp.take(kv, col, axis=0)

    P = jnp.repeat(Wexp.T, HEAD_DIM, axis=0)
    R = jnp.repeat(jnp.eye(EXP_HEADS, dtype=jnp.float32), HEAD_DIM, axis=1)

    BE = 2000
    msg, exb = pl.pallas_call(
        functools.partial(_edge_body, scale=scale),
        grid=(E // BE,),
        in_specs=[
            pl.BlockSpec((BE, DIM), lambda i: (i, 0)),
            pl.BlockSpec((BE, DIM), lambda i: (i, 0)),
            pl.BlockSpec((BE, 2 * DIM), lambda i: (i, 0)),
            pl.BlockSpec((DIM, DIM), lambda i: (0, 0)),
            pl.BlockSpec((DIM, DIM), lambda i: (0, 0)),
            pl.BlockSpec((DIM, EXP_HEADS), lambda i: (0, 0)),
            pl.BlockSpec((EXP_HEADS, DIM), lambda i: (0, 0)),
        ],
        out_specs=[
            pl.BlockSpec((BE, DIM), lambda i: (i, 0)),
            pl.BlockSpec((BE, DIM), lambda i: (i, 0)),
        ],
        out_shape=[
            jax.ShapeDtypeStruct((E, DIM), jnp.float32),
            jax.ShapeDtypeStruct((E, DIM), jnp.float32),
        ],
    )(edges, qe, kve, Wek, Wev, P, R)

    row2d = row.reshape(NGRP, 128)
    zros = jnp.zeros((NP, DIM), jnp.float32)
    num2, den2 = pl.kernel(
        _scatter_body,
        out_type=[
            jax.ShapeDtypeStruct((2, NP, DIM), jnp.float32),
            jax.ShapeDtypeStruct((2, NP, DIM), jnp.float32),
        ],
        mesh=plsc.VectorSubcoreMesh(core_axis_name="c", subcore_axis_name="s"),
        scratch_types=[
            pltpu.VMEM((1, 128), jnp.int32),
            pltpu.VMEM((128, DIM), jnp.float32),
            pltpu.VMEM_SHARED((NP, DIM), jnp.float32),
        ],
    )(msg, exb, row2d, zros)

    out = pl.pallas_call(
        _finish_body,
        grid=(N // BN,),
        in_specs=[
            pl.BlockSpec((BN, DIM), lambda i: (i, 0)),
            pl.BlockSpec((2, BN, DIM), lambda i: (0, i, 0)),
            pl.BlockSpec((2, BN, DIM), lambda i: (0, i, 0)),
            pl.BlockSpec((DIM, DIM), lambda i: (0, 0)),
        ],
        out_specs=pl.BlockSpec((BN, DIM), lambda i: (i, 0)),
        out_shape=jax.ShapeDtypeStruct((N, DIM), jnp.float32),
    )(vfull, num2, den2, Wout)
    return out


# speedup vs baseline: 39.5329x; 2.0773x over previous
import functools

import jax
import jax.numpy as jnp
from jax import lax
from jax.experimental import pallas as pl
from jax.experimental.pallas import tpu as pltpu
from jax.experimental.pallas import tpu_sc as plsc

N = 10000
E = 320000
DIM = 128
HEADS = 8
HEAD_DIM = DIM // HEADS
EXP_HEADS = 8

NGRP = E // 128
GRP_PER_SC = NGRP // 2
GRP_BASE = GRP_PER_SC // 16
GRP_REM = GRP_PER_SC % 16
ROWS_PER_TILE = 632
NP = 16 * ROWS_PER_TILE


def _proj_body(x_ref, wq_ref, wk_ref, wv_ref, q_ref, kv_ref, v_ref, *, scale):
    x = x_ref[...]
    q_ref[...] = jnp.dot(x, wq_ref[...].T, preferred_element_type=jnp.float32)
    kv_ref[:, :DIM] = jnp.dot(x, wk_ref[...].T, preferred_element_type=jnp.float32) * scale
    v = jnp.dot(x, wv_ref[...].T, preferred_element_type=jnp.float32)
    kv_ref[:, DIM:] = v
    v_ref[...] = v


def _edge_body(e_ref, qe_ref, kve_ref, wek_ref, wev_ref, p_ref, r_ref,
               msg_ref, exb_ref, *, scale):
    e = e_ref[...]
    ek = jnp.dot(e, wek_ref[...].T, preferred_element_type=jnp.float32) * scale
    ev = jnp.dot(e, wev_ref[...].T, preferred_element_type=jnp.float32)
    qe = qe_ref[...]
    kve = kve_ref[...]
    ke = kve[:, :DIM] + ek
    scores = jnp.dot(qe * ke, p_ref[...], preferred_element_type=jnp.float32)
    ex = jnp.exp(jnp.minimum(scores, 75.0))
    ex128 = jnp.dot(ex, r_ref[...], preferred_element_type=jnp.float32)
    ve = kve[:, DIM:] + ev
    msg_ref[...] = ex128 * ve
    exb_ref[...] = ex128


GW_BLK = 80
NGRP_PAD = 32 * GW_BLK


def _gather_body(q_hbm, kv_hbm, rowp_hbm, colp_hbm, qe_out, kve_out,
                 idxr_v, idxc_v, stq_v, stkv_v, semq, semkv):
    c = lax.axis_index("c")
    s = lax.axis_index("s")
    wid = c * 16 + s
    base = wid * GW_BLK
    count = jnp.minimum(GW_BLK, NGRP - base)
    pltpu.sync_copy(rowp_hbm.at[pl.ds(base, GW_BLK)], idxr_v)
    pltpu.sync_copy(colp_hbm.at[pl.ds(base, GW_BLK)], idxc_v)

    def body(i, carry):
        g = base + i
        cp1 = pltpu.async_copy(q_hbm.at[idxr_v.at[i]], stq_v, semq)
        cp2 = pltpu.async_copy(kv_hbm.at[idxc_v.at[i]], stkv_v, semkv)
        cp1.wait()
        cp2.wait()
        o1 = pltpu.async_copy(stq_v, qe_out.at[pl.ds(g * 128, 128)], semq)
        o2 = pltpu.async_copy(stkv_v, kve_out.at[pl.ds(g * 128, 128)], semkv)
        o1.wait()
        o2.wait()
        return carry

    lax.fori_loop(0, count, body, 0)


def _scatter_body(msg_hbm, exb_hbm, row_hbm, zro_hbm,
                  num_out, den_out,
                  idx2_v, stage_v, acc):
    c = lax.axis_index("c")
    s = lax.axis_index("s")
    base = c * GRP_PER_SC + s * GRP_BASE + jnp.minimum(s, GRP_REM)
    count = GRP_BASE + (s < GRP_REM).astype(jnp.int32)
    rslice = pl.ds(s * ROWS_PER_TILE, ROWS_PER_TILE)

    def phase(data_hbm, out_hbm):
        pltpu.sync_copy(zro_hbm.at[rslice], acc.at[rslice])
        plsc.subcore_barrier()

        def group_body(i, carry):
            g = base + i
            pltpu.sync_copy(row_hbm.at[pl.ds(g, 1)], idx2_v)
            pltpu.sync_copy(data_hbm.at[pl.ds(g * 128, 128)], stage_v)
            pltpu.sync_copy(stage_v, acc.at[idx2_v.at[0]], add=True)
            return carry

        lax.fori_loop(0, count, group_body, 0)
        plsc.subcore_barrier()
        pltpu.sync_copy(acc.at[rslice], out_hbm.at[c, rslice])
        plsc.subcore_barrier()

    phase(msg_hbm, num_out)
    phase(exb_hbm, den_out)


def _finish_body(v_ref, num_ref, den_ref, wout_ref, o_ref):
    v = v_ref[...]
    num = num_ref[0] + num_ref[1]
    den = den_ref[0] + den_ref[1]
    out = v - num / (den + 1e-9)
    o_ref[...] = jnp.dot(out, wout_ref[...].T, preferred_element_type=jnp.float32)


def kernel(x, edges, edge_index, Wq, Wk, Wv, Wek, Wev, Wexp, Wout):
    scale = HEAD_DIM ** -0.5
    BN = 1000
    q, kv, vfull = pl.pallas_call(
        functools.partial(_proj_body, scale=scale),
        grid=(N // BN,),
        in_specs=[
            pl.BlockSpec((BN, DIM), lambda i: (i, 0)),
            pl.BlockSpec((DIM, DIM), lambda i: (0, 0)),
            pl.BlockSpec((DIM, DIM), lambda i: (0, 0)),
            pl.BlockSpec((DIM, DIM), lambda i: (0, 0)),
        ],
        out_specs=[
            pl.BlockSpec((BN, DIM), lambda i: (i, 0)),
            pl.BlockSpec((BN, 2 * DIM), lambda i: (i, 0)),
            pl.BlockSpec((BN, DIM), lambda i: (i, 0)),
        ],
        out_shape=[
            jax.ShapeDtypeStruct((N, DIM), jnp.float32),
            jax.ShapeDtypeStruct((N, 2 * DIM), jnp.float32),
            jax.ShapeDtypeStruct((N, DIM), jnp.float32),
        ],
    )(x, Wq, Wk, Wv)

    row = edge_index[0]
    col = edge_index[1]
    zpad = jnp.zeros((NGRP_PAD - NGRP, 128), jnp.int32)
    rowp = jnp.concatenate([row.reshape(NGRP, 128), zpad])
    colp = jnp.concatenate([col.reshape(NGRP, 128), zpad])
    qe, kve = pl.kernel(
        _gather_body,
        out_type=[
            jax.ShapeDtypeStruct((E, DIM), jnp.float32),
            jax.ShapeDtypeStruct((E, 2 * DIM), jnp.float32),
        ],
        mesh=plsc.VectorSubcoreMesh(core_axis_name="c", subcore_axis_name="s"),
        scratch_types=[
            pltpu.VMEM((GW_BLK, 128), jnp.int32),
            pltpu.VMEM((GW_BLK, 128), jnp.int32),
            pltpu.VMEM((128, DIM), jnp.float32),
            pltpu.VMEM((128, 2 * DIM), jnp.float32),
            pltpu.SemaphoreType.DMA,
            pltpu.SemaphoreType.DMA,
        ],
    )(q, kv, rowp, colp)

    P = jnp.repeat(Wexp.T, HEAD_DIM, axis=0)
    R = jnp.repeat(jnp.eye(EXP_HEADS, dtype=jnp.float32), HEAD_DIM, axis=1)

    BE = 2000
    msg, exb = pl.pallas_call(
        functools.partial(_edge_body, scale=scale),
        grid=(E // BE,),
        in_specs=[
            pl.BlockSpec((BE, DIM), lambda i: (i, 0)),
            pl.BlockSpec((BE, DIM), lambda i: (i, 0)),
            pl.BlockSpec((BE, 2 * DIM), lambda i: (i, 0)),
            pl.BlockSpec((DIM, DIM), lambda i: (0, 0)),
            pl.BlockSpec((DIM, DIM), lambda i: (0, 0)),
            pl.BlockSpec((DIM, EXP_HEADS), lambda i: (0, 0)),
            pl.BlockSpec((EXP_HEADS, DIM), lambda i: (0, 0)),
        ],
        out_specs=[
            pl.BlockSpec((BE, DIM), lambda i: (i, 0)),
            pl.BlockSpec((BE, DIM), lambda i: (i, 0)),
        ],
        out_shape=[
            jax.ShapeDtypeStruct((E, DIM), jnp.float32),
            jax.ShapeDtypeStruct((E, DIM), jnp.float32),
        ],
    )(edges, qe, kve, Wek, Wev, P, R)

    row2d = rowp[:NGRP]
    zros = jnp.zeros((NP, DIM), jnp.float32)
    num2, den2 = pl.kernel(
        _scatter_body,
        out_type=[
            jax.ShapeDtypeStruct((2, NP, DIM), jnp.float32),
            jax.ShapeDtypeStruct((2, NP, DIM), jnp.float32),
        ],
        mesh=plsc.VectorSubcoreMesh(core_axis_name="c", subcore_axis_name="s"),
        scratch_types=[
            pltpu.VMEM((1, 128), jnp.int32),
            pltpu.VMEM((128, DIM), jnp.float32),
            pltpu.VMEM_SHARED((NP, DIM), jnp.float32),
        ],
    )(msg, exb, row2d, zros)

    out = pl.pallas_call(
        _finish_body,
        grid=(N // BN,),
        in_specs=[
            pl.BlockSpec((BN, DIM), lambda i: (i, 0)),
            pl.BlockSpec((2, BN, DIM), lambda i: (0, i, 0)),
            pl.BlockSpec((2, BN, DIM), lambda i: (0, i, 0)),
            pl.BlockSpec((DIM, DIM), lambda i: (0, 0)),
        ],
        out_specs=pl.BlockSpec((BN, DIM), lambda i: (i, 0)),
        out_shape=jax.ShapeDtypeStruct((N, DIM), jnp.float32),
    )(vfull, num2, den2, Wout)
    return out


# speedup vs baseline: 45.2854x; 1.1455x over previous
import functools

import jax
import jax.numpy as jnp
from jax import lax
from jax.experimental import pallas as pl
from jax.experimental.pallas import tpu as pltpu
from jax.experimental.pallas import tpu_sc as plsc

N = 10000
E = 320000
DIM = 128
HEADS = 8
HEAD_DIM = DIM // HEADS
EXP_HEADS = 8

NGRP = E // 128
GRP_PER_SC = NGRP // 2
GRP_BASE = GRP_PER_SC // 16
GRP_REM = GRP_PER_SC % 16
ROWS_PER_TILE = 632
NP = 16 * ROWS_PER_TILE


def _proj_body(x_ref, wq_ref, wk_ref, wv_ref, q_ref, kv_ref, v_ref, *, scale):
    x = x_ref[...]
    q_ref[...] = jnp.dot(x, wq_ref[...].T, preferred_element_type=jnp.float32)
    kv_ref[:, :DIM] = jnp.dot(x, wk_ref[...].T, preferred_element_type=jnp.float32) * scale
    v = jnp.dot(x, wv_ref[...].T, preferred_element_type=jnp.float32)
    kv_ref[:, DIM:] = v
    v_ref[...] = v


def _edge_body(e_ref, qe_ref, kve_ref, wek_ref, wev_ref, p_ref, r_ref,
               msg_ref, exb_ref, *, scale):
    e = e_ref[...]
    ek = jnp.dot(e, wek_ref[...].T, preferred_element_type=jnp.float32) * scale
    ev = jnp.dot(e, wev_ref[...].T, preferred_element_type=jnp.float32)
    qe = qe_ref[...]
    kve = kve_ref[...]
    ke = kve[:, :DIM] + ek
    scores = jnp.dot(qe * ke, p_ref[...], preferred_element_type=jnp.float32)
    ex = jnp.exp(jnp.minimum(scores, 75.0))
    ex128 = jnp.dot(ex, r_ref[...], preferred_element_type=jnp.float32)
    ve = kve[:, DIM:] + ev
    msg_ref[...] = ex128 * ve
    exb_ref[...] = ex128


GW_BLK = 80
NGRP_PAD = 32 * GW_BLK


def _gather_body(q_hbm, kv_hbm, rowp_hbm, colp_hbm, qe_out, kve_out,
                 idxr_v, idxc_v, stq_v, stkv_v, semq, semkv):
    c = lax.axis_index("c")
    s = lax.axis_index("s")
    wid = c * 16 + s
    base = wid * GW_BLK
    count = jnp.minimum(GW_BLK, NGRP - base)
    pltpu.sync_copy(rowp_hbm.at[pl.ds(base, GW_BLK)], idxr_v)
    pltpu.sync_copy(colp_hbm.at[pl.ds(base, GW_BLK)], idxc_v)

    def body(i, carry):
        g = base + i
        cp1 = pltpu.async_copy(q_hbm.at[idxr_v.at[i]], stq_v, semq)
        cp2 = pltpu.async_copy(kv_hbm.at[idxc_v.at[i]], stkv_v, semkv)
        cp1.wait()
        o1 = pltpu.async_copy(stq_v, qe_out.at[pl.ds(g * 128, 128)], semq)
        cp2.wait()
        o2 = pltpu.async_copy(stkv_v, kve_out.at[pl.ds(g * 128, 128)], semkv)
        o1.wait()
        o2.wait()
        return carry

    lax.fori_loop(0, count, body, 0)


def _scatter_body(msg_hbm, exb_hbm, rowp_hbm, zro_hbm,
                  num_out, den_out,
                  idxs_v, st0_v, st1_v, acc, sem_a, sem_b):
    c = lax.axis_index("c")
    s = lax.axis_index("s")
    wid = c * 16 + s
    base = wid * GW_BLK
    count = jnp.minimum(GW_BLK, NGRP - base)
    rslice = pl.ds(s * ROWS_PER_TILE, ROWS_PER_TILE)
    pltpu.sync_copy(rowp_hbm.at[pl.ds(base, GW_BLK)], idxs_v)

    def phase(data_hbm, out_hbm):
        pltpu.sync_copy(zro_hbm.at[rslice], acc.at[rslice])
        plsc.subcore_barrier()

        def body(j, carry):
            g0 = (base + 2 * j) * 128
            c0 = pltpu.async_copy(data_hbm.at[pl.ds(g0, 128)], st0_v, sem_a)
            c1 = pltpu.async_copy(data_hbm.at[pl.ds(g0 + 128, 128)], st1_v, sem_b)
            c0.wait()
            s0 = pltpu.async_copy(st0_v, acc.at[idxs_v.at[2 * j]], sem_a, add=True)
            c1.wait()
            s1 = pltpu.async_copy(st1_v, acc.at[idxs_v.at[2 * j + 1]], sem_b, add=True)
            s0.wait()
            s1.wait()
            return carry

        lax.fori_loop(0, count // 2, body, 0)
        plsc.subcore_barrier()
        pltpu.sync_copy(acc.at[rslice], out_hbm.at[c, rslice])
        plsc.subcore_barrier()

    phase(msg_hbm, num_out)
    phase(exb_hbm, den_out)


def _finish_body(v_ref, num_ref, den_ref, wout_ref, o_ref):
    v = v_ref[...]
    num = num_ref[0] + num_ref[1]
    den = den_ref[0] + den_ref[1]
    out = v - num / (den + 1e-9)
    o_ref[...] = jnp.dot(out, wout_ref[...].T, preferred_element_type=jnp.float32)


def kernel(x, edges, edge_index, Wq, Wk, Wv, Wek, Wev, Wexp, Wout):
    scale = HEAD_DIM ** -0.5
    BN = 1000
    q, kv, vfull = pl.pallas_call(
        functools.partial(_proj_body, scale=scale),
        grid=(N // BN,),
        in_specs=[
            pl.BlockSpec((BN, DIM), lambda i: (i, 0)),
            pl.BlockSpec((DIM, DIM), lambda i: (0, 0)),
            pl.BlockSpec((DIM, DIM), lambda i: (0, 0)),
            pl.BlockSpec((DIM, DIM), lambda i: (0, 0)),
        ],
        out_specs=[
            pl.BlockSpec((BN, DIM), lambda i: (i, 0)),
            pl.BlockSpec((BN, 2 * DIM), lambda i: (i, 0)),
            pl.BlockSpec((BN, DIM), lambda i: (i, 0)),
        ],
        out_shape=[
            jax.ShapeDtypeStruct((N, DIM), jnp.float32),
            jax.ShapeDtypeStruct((N, 2 * DIM), jnp.float32),
            jax.ShapeDtypeStruct((N, DIM), jnp.float32),
        ],
    )(x, Wq, Wk, Wv)

    row = edge_index[0]
    col = edge_index[1]
    zpad = jnp.zeros((NGRP_PAD - NGRP, 128), jnp.int32)
    rowp = jnp.concatenate([row.reshape(NGRP, 128), zpad])
    colp = jnp.concatenate([col.reshape(NGRP, 128), zpad])
    qe, kve = pl.kernel(
        _gather_body,
        out_type=[
            jax.ShapeDtypeStruct((E, DIM), jnp.float32),
            jax.ShapeDtypeStruct((E, 2 * DIM), jnp.float32),
        ],
        mesh=plsc.VectorSubcoreMesh(core_axis_name="c", subcore_axis_name="s"),
        scratch_types=[
            pltpu.VMEM((GW_BLK, 128), jnp.int32),
            pltpu.VMEM((GW_BLK, 128), jnp.int32),
            pltpu.VMEM((128, DIM), jnp.float32),
            pltpu.VMEM((128, 2 * DIM), jnp.float32),
            pltpu.SemaphoreType.DMA,
            pltpu.SemaphoreType.DMA,
        ],
    )(q, kv, rowp, colp)

    P = jnp.repeat(Wexp.T, HEAD_DIM, axis=0)
    R = jnp.repeat(jnp.eye(EXP_HEADS, dtype=jnp.float32), HEAD_DIM, axis=1)

    BE = 2000
    msg, exb = pl.pallas_call(
        functools.partial(_edge_body, scale=scale),
        grid=(E // BE,),
        in_specs=[
            pl.BlockSpec((BE, DIM), lambda i: (i, 0)),
            pl.BlockSpec((BE, DIM), lambda i: (i, 0)),
            pl.BlockSpec((BE, 2 * DIM), lambda i: (i, 0)),
            pl.BlockSpec((DIM, DIM), lambda i: (0, 0)),
            pl.BlockSpec((DIM, DIM), lambda i: (0, 0)),
            pl.BlockSpec((DIM, EXP_HEADS), lambda i: (0, 0)),
            pl.BlockSpec((EXP_HEADS, DIM), lambda i: (0, 0)),
        ],
        out_specs=[
            pl.BlockSpec((BE, DIM), lambda i: (i, 0)),
            pl.BlockSpec((BE, DIM), lambda i: (i, 0)),
        ],
        out_shape=[
            jax.ShapeDtypeStruct((E, DIM), jnp.float32),
            jax.ShapeDtypeStruct((E, DIM), jnp.float32),
        ],
    )(edges, qe, kve, Wek, Wev, P, R)

    zros = jnp.zeros((NP, DIM), jnp.float32)
    num2, den2 = pl.kernel(
        _scatter_body,
        out_type=[
            jax.ShapeDtypeStruct((2, NP, DIM), jnp.float32),
            jax.ShapeDtypeStruct((2, NP, DIM), jnp.float32),
        ],
        mesh=plsc.VectorSubcoreMesh(core_axis_name="c", subcore_axis_name="s"),
        scratch_types=[
            pltpu.VMEM((GW_BLK, 128), jnp.int32),
            pltpu.VMEM((128, DIM), jnp.float32),
            pltpu.VMEM((128, DIM), jnp.float32),
            pltpu.VMEM_SHARED((NP, DIM), jnp.float32),
            pltpu.SemaphoreType.DMA,
            pltpu.SemaphoreType.DMA,
        ],
    )(msg, exb, rowp, zros)

    out = pl.pallas_call(
        _finish_body,
        grid=(N // BN,),
        in_specs=[
            pl.BlockSpec((BN, DIM), lambda i: (i, 0)),
            pl.BlockSpec((2, BN, DIM), lambda i: (0, i, 0)),
            pl.BlockSpec((2, BN, DIM), lambda i: (0, i, 0)),
            pl.BlockSpec((DIM, DIM), lambda i: (0, 0)),
        ],
        out_specs=pl.BlockSpec((BN, DIM), lambda i: (i, 0)),
        out_shape=jax.ShapeDtypeStruct((N, DIM), jnp.float32),
    )(vfull, num2, den2, Wout)
    return out


# speedup vs baseline: 50.9521x; 1.1251x over previous
import functools

import jax
import jax.numpy as jnp
from jax import lax
from jax.experimental import pallas as pl
from jax.experimental.pallas import tpu as pltpu
from jax.experimental.pallas import tpu_sc as plsc

N = 10000
E = 320000
DIM = 128
HEADS = 8
HEAD_DIM = DIM // HEADS
EXP_HEADS = 8

NGRP = E // 128
GRP_PER_SC = NGRP // 2
GRP_BASE = GRP_PER_SC // 16
GRP_REM = GRP_PER_SC % 16
ROWS_PER_TILE = 632
NP = 16 * ROWS_PER_TILE


def _pack_halves(a, b):
    lo = lax.bitcast_convert_type(a.astype(jnp.bfloat16), jnp.uint16).astype(jnp.uint32)
    hi = lax.bitcast_convert_type(b.astype(jnp.bfloat16), jnp.uint16).astype(jnp.uint32)
    return lax.bitcast_convert_type(lo | (hi << 16), jnp.int32)


def _unpack_halves(p):
    u = lax.bitcast_convert_type(p, jnp.uint32)
    lo = lax.bitcast_convert_type((u & 0xFFFF).astype(jnp.uint16), jnp.bfloat16)
    hi = lax.bitcast_convert_type((u >> 16).astype(jnp.uint16), jnp.bfloat16)
    return lo.astype(jnp.float32), hi.astype(jnp.float32)


def _proj_body(x_ref, wq_ref, wk_ref, wv_ref, q_ref, kv_ref, v_ref, *, scale):
    x = x_ref[...]
    q_ref[...] = jnp.dot(x, wq_ref[...].T, preferred_element_type=jnp.float32)
    k = jnp.dot(x, wk_ref[...].T, preferred_element_type=jnp.float32) * scale
    v = jnp.dot(x, wv_ref[...].T, preferred_element_type=jnp.float32)
    kv_ref[...] = _pack_halves(k, v)
    v_ref[...] = v


def _edge_body(e_ref, qe_ref, kve_ref, wek_ref, wev_ref, p_ref, r_ref,
               msg_ref, exb_ref, *, scale):
    e = e_ref[...]
    ek = jnp.dot(e, wek_ref[...].T, preferred_element_type=jnp.float32) * scale
    ev = jnp.dot(e, wev_ref[...].T, preferred_element_type=jnp.float32)
    qe = qe_ref[...]
    kg, vg = _unpack_halves(kve_ref[...])
    ke = kg + ek
    scores = jnp.dot(qe * ke, p_ref[...], preferred_element_type=jnp.float32)
    ex = jnp.exp(jnp.minimum(scores, 75.0))
    ex128 = jnp.dot(ex, r_ref[...], preferred_element_type=jnp.float32)
    ve = vg + ev
    msg_ref[...] = ex128 * ve
    exb_ref[...] = ex128


GW_BLK = 80
NGRP_PAD = 32 * GW_BLK


def _gather_body(q_hbm, kv_hbm, rowp_hbm, colp_hbm, qe_out, kve_out,
                 idxr_v, idxc_v, stq_v, stkv_v, semq, semkv):
    c = lax.axis_index("c")
    s = lax.axis_index("s")
    wid = c * 16 + s
    base = wid * GW_BLK
    count = jnp.minimum(GW_BLK, NGRP - base)
    pltpu.sync_copy(rowp_hbm.at[pl.ds(base, GW_BLK)], idxr_v)
    pltpu.sync_copy(colp_hbm.at[pl.ds(base, GW_BLK)], idxc_v)

    def body(i, carry):
        g = base + i
        cp1 = pltpu.async_copy(q_hbm.at[idxr_v.at[i]], stq_v, semq)
        cp2 = pltpu.async_copy(kv_hbm.at[idxc_v.at[i]], stkv_v, semkv)
        cp1.wait()
        o1 = pltpu.async_copy(stq_v, qe_out.at[pl.ds(g * 128, 128)], semq)
        cp2.wait()
        o2 = pltpu.async_copy(stkv_v, kve_out.at[pl.ds(g * 128, 128)], semkv)
        o1.wait()
        o2.wait()
        return carry

    lax.fori_loop(0, count, body, 0)


def _scatter_body(msg_hbm, exb_hbm, rowp_hbm, zro_hbm,
                  num_out, den_out,
                  idxs_v, st0_v, st1_v, acc, sem_a, sem_b):
    c = lax.axis_index("c")
    s = lax.axis_index("s")
    wid = c * 16 + s
    base = wid * GW_BLK
    count = jnp.minimum(GW_BLK, NGRP - base)
    rslice = pl.ds(s * ROWS_PER_TILE, ROWS_PER_TILE)
    pltpu.sync_copy(rowp_hbm.at[pl.ds(base, GW_BLK)], idxs_v)

    def phase(data_hbm, out_hbm):
        pltpu.sync_copy(zro_hbm.at[rslice], acc.at[rslice])
        plsc.subcore_barrier()

        def body(j, carry):
            g0 = (base + 2 * j) * 128
            c0 = pltpu.async_copy(data_hbm.at[pl.ds(g0, 128)], st0_v, sem_a)
            c1 = pltpu.async_copy(data_hbm.at[pl.ds(g0 + 128, 128)], st1_v, sem_b)
            c0.wait()
            s0 = pltpu.async_copy(st0_v, acc.at[idxs_v.at[2 * j]], sem_a, add=True)
            c1.wait()
            s1 = pltpu.async_copy(st1_v, acc.at[idxs_v.at[2 * j + 1]], sem_b, add=True)
            s0.wait()
            s1.wait()
            return carry

        lax.fori_loop(0, count // 2, body, 0)
        plsc.subcore_barrier()
        pltpu.sync_copy(acc.at[rslice], out_hbm.at[c, rslice])
        plsc.subcore_barrier()

    phase(msg_hbm, num_out)
    phase(exb_hbm, den_out)


def _finish_body(v_ref, num_ref, den_ref, wout_ref, o_ref):
    v = v_ref[...]
    num = num_ref[0] + num_ref[1]
    den = den_ref[0] + den_ref[1]
    out = v - num / (den + 1e-9)
    o_ref[...] = jnp.dot(out, wout_ref[...].T, preferred_element_type=jnp.float32)


def kernel(x, edges, edge_index, Wq, Wk, Wv, Wek, Wev, Wexp, Wout):
    scale = HEAD_DIM ** -0.5
    BN = 1000
    q, kv, vfull = pl.pallas_call(
        functools.partial(_proj_body, scale=scale),
        grid=(N // BN,),
        in_specs=[
            pl.BlockSpec((BN, DIM), lambda i: (i, 0)),
            pl.BlockSpec((DIM, DIM), lambda i: (0, 0)),
            pl.BlockSpec((DIM, DIM), lambda i: (0, 0)),
            pl.BlockSpec((DIM, DIM), lambda i: (0, 0)),
        ],
        out_specs=[
            pl.BlockSpec((BN, DIM), lambda i: (i, 0)),
            pl.BlockSpec((BN, DIM), lambda i: (i, 0)),
            pl.BlockSpec((BN, DIM), lambda i: (i, 0)),
        ],
        out_shape=[
            jax.ShapeDtypeStruct((N, DIM), jnp.float32),
            jax.ShapeDtypeStruct((N, DIM), jnp.int32),
            jax.ShapeDtypeStruct((N, DIM), jnp.float32),
        ],
    )(x, Wq, Wk, Wv)

    row = edge_index[0]
    col = edge_index[1]
    zpad = jnp.zeros((NGRP_PAD - NGRP, 128), jnp.int32)
    rowp = jnp.concatenate([row.reshape(NGRP, 128), zpad])
    colp = jnp.concatenate([col.reshape(NGRP, 128), zpad])
    qe, kve = pl.kernel(
        _gather_body,
        out_type=[
            jax.ShapeDtypeStruct((E, DIM), jnp.float32),
            jax.ShapeDtypeStruct((E, DIM), jnp.int32),
        ],
        mesh=plsc.VectorSubcoreMesh(core_axis_name="c", subcore_axis_name="s"),
        scratch_types=[
            pltpu.VMEM((GW_BLK, 128), jnp.int32),
            pltpu.VMEM((GW_BLK, 128), jnp.int32),
            pltpu.VMEM((128, DIM), jnp.float32),
            pltpu.VMEM((128, DIM), jnp.int32),
            pltpu.SemaphoreType.DMA,
            pltpu.SemaphoreType.DMA,
        ],
    )(q, kv, rowp, colp)

    P = jnp.repeat(Wexp.T, HEAD_DIM, axis=0)
    R = jnp.repeat(jnp.eye(EXP_HEADS, dtype=jnp.float32), HEAD_DIM, axis=1)

    BE = 2000
    msg, exb = pl.pallas_call(
        functools.partial(_edge_body, scale=scale),
        grid=(E // BE,),
        in_specs=[
            pl.BlockSpec((BE, DIM), lambda i: (i, 0)),
            pl.BlockSpec((BE, DIM), lambda i: (i, 0)),
            pl.BlockSpec((BE, DIM), lambda i: (i, 0)),
            pl.BlockSpec((DIM, DIM), lambda i: (0, 0)),
            pl.BlockSpec((DIM, DIM), lambda i: (0, 0)),
            pl.BlockSpec((DIM, EXP_HEADS), lambda i: (0, 0)),
            pl.BlockSpec((EXP_HEADS, DIM), lambda i: (0, 0)),
        ],
        out_specs=[
            pl.BlockSpec((BE, DIM), lambda i: (i, 0)),
            pl.BlockSpec((BE, DIM), lambda i: (i, 0)),
        ],
        out_shape=[
            jax.ShapeDtypeStruct((E, DIM), jnp.float32),
            jax.ShapeDtypeStruct((E, DIM), jnp.float32),
        ],
    )(edges, qe, kve, Wek, Wev, P, R)

    zros = jnp.zeros((NP, DIM), jnp.float32)
    num2, den2 = pl.kernel(
        _scatter_body,
        out_type=[
            jax.ShapeDtypeStruct((2, NP, DIM), jnp.float32),
            jax.ShapeDtypeStruct((2, NP, DIM), jnp.float32),
        ],
        mesh=plsc.VectorSubcoreMesh(core_axis_name="c", subcore_axis_name="s"),
        scratch_types=[
            pltpu.VMEM((GW_BLK, 128), jnp.int32),
            pltpu.VMEM((128, DIM), jnp.float32),
            pltpu.VMEM((128, DIM), jnp.float32),
            pltpu.VMEM_SHARED((NP, DIM), jnp.float32),
            pltpu.SemaphoreType.DMA,
            pltpu.SemaphoreType.DMA,
        ],
    )(msg, exb, rowp, zros)

    out = pl.pallas_call(
        _finish_body,
        grid=(N // BN,),
        in_specs=[
            pl.BlockSpec((BN, DIM), lambda i: (i, 0)),
            pl.BlockSpec((2, BN, DIM), lambda i: (0, i, 0)),
            pl.BlockSpec((2, BN, DIM), lambda i: (0, i, 0)),
            pl.BlockSpec((DIM, DIM), lambda i: (0, 0)),
        ],
        out_specs=pl.BlockSpec((BN, DIM), lambda i: (i, 0)),
        out_shape=jax.ShapeDtypeStruct((N, DIM), jnp.float32),
    )(vfull, num2, den2, Wout)
    return out


# speedup vs baseline: 52.1554x; 1.0236x over previous
import functools

import jax
import jax.numpy as jnp
from jax import lax
from jax.experimental import pallas as pl
from jax.experimental.pallas import tpu as pltpu
from jax.experimental.pallas import tpu_sc as plsc

N = 10000
E = 320000
DIM = 128
HEADS = 8
HEAD_DIM = DIM // HEADS
EXP_HEADS = 8

NGRP = E // 128
GRP_PER_SC = NGRP // 2
GRP_BASE = GRP_PER_SC // 16
GRP_REM = GRP_PER_SC % 16
ROWS_PER_TILE = 632
NP = 16 * ROWS_PER_TILE


def _pack_halves(a, b):
    lo = lax.bitcast_convert_type(a.astype(jnp.bfloat16), jnp.uint16).astype(jnp.uint32)
    hi = lax.bitcast_convert_type(b.astype(jnp.bfloat16), jnp.uint16).astype(jnp.uint32)
    return lax.bitcast_convert_type(lo | (hi << 16), jnp.int32)


def _unpack_halves(p):
    u = lax.bitcast_convert_type(p, jnp.uint32)
    lo = lax.bitcast_convert_type((u & 0xFFFF).astype(jnp.uint16), jnp.bfloat16)
    hi = lax.bitcast_convert_type((u >> 16).astype(jnp.uint16), jnp.bfloat16)
    return lo.astype(jnp.float32), hi.astype(jnp.float32)


def _proj_body(x_ref, wq_ref, wk_ref, wv_ref, q_ref, kv_ref, v_ref, *, scale):
    x = x_ref[...]
    q_ref[...] = jnp.dot(x, wq_ref[...].T, preferred_element_type=jnp.float32)
    k = jnp.dot(x, wk_ref[...].T, preferred_element_type=jnp.float32) * scale
    v = jnp.dot(x, wv_ref[...].T, preferred_element_type=jnp.float32)
    kv_ref[...] = _pack_halves(k, v)
    v_ref[...] = v


def _edge_body(e_ref, qe_ref, kve_ref, wek_ref, wev_ref, p_ref, r_ref,
               msg_ref, exb_ref, *, scale):
    e = e_ref[...]
    ek = jnp.dot(e, wek_ref[...].T, preferred_element_type=jnp.float32) * scale
    ev = jnp.dot(e, wev_ref[...].T, preferred_element_type=jnp.float32)
    qe = qe_ref[...]
    kg, vg = _unpack_halves(kve_ref[...])
    ke = kg + ek
    scores = jnp.dot(qe * ke, p_ref[...], preferred_element_type=jnp.float32)
    ex = jnp.exp(jnp.minimum(scores, 75.0))
    ex128 = jnp.dot(ex, r_ref[...], preferred_element_type=jnp.float32)
    ve = vg + ev
    msg_ref[...] = ex128 * ve
    exb_ref[...] = ex128


GW_BLK = 80
NGRP_PAD = 32 * GW_BLK


def _gather_body(q_hbm, kv_hbm, rowp_hbm, colp_hbm, qe_out, kve_out,
                 idxr_v, idxc_v, stq_v, stkv_v, stq2_v, stkv2_v,
                 semq, semkv, semq2, semkv2):
    c = lax.axis_index("c")
    s = lax.axis_index("s")
    wid = c * 16 + s
    base = wid * GW_BLK
    count = jnp.minimum(GW_BLK, NGRP - base)
    pltpu.sync_copy(rowp_hbm.at[pl.ds(base, GW_BLK)], idxr_v)
    pltpu.sync_copy(colp_hbm.at[pl.ds(base, GW_BLK)], idxc_v)

    def body(j, carry):
        i0 = 2 * j
        g0 = (base + i0) * 128
        cp1 = pltpu.async_copy(q_hbm.at[idxr_v.at[i0]], stq_v, semq)
        cp2 = pltpu.async_copy(kv_hbm.at[idxc_v.at[i0]], stkv_v, semkv)
        cp3 = pltpu.async_copy(q_hbm.at[idxr_v.at[i0 + 1]], stq2_v, semq2)
        cp4 = pltpu.async_copy(kv_hbm.at[idxc_v.at[i0 + 1]], stkv2_v, semkv2)
        cp1.wait()
        o1 = pltpu.async_copy(stq_v, qe_out.at[pl.ds(g0, 128)], semq)
        cp2.wait()
        o2 = pltpu.async_copy(stkv_v, kve_out.at[pl.ds(g0, 128)], semkv)
        cp3.wait()
        o3 = pltpu.async_copy(stq2_v, qe_out.at[pl.ds(g0 + 128, 128)], semq2)
        cp4.wait()
        o4 = pltpu.async_copy(stkv2_v, kve_out.at[pl.ds(g0 + 128, 128)], semkv2)
        o1.wait()
        o2.wait()
        o3.wait()
        o4.wait()
        return carry

    lax.fori_loop(0, count // 2, body, 0)


def _scatter_body(msg_hbm, exb_hbm, rowp_hbm, zro_hbm,
                  num_out, den_out,
                  idxs_v, st0_v, st1_v, acc, sem_a, sem_b):
    c = lax.axis_index("c")
    s = lax.axis_index("s")
    wid = c * 16 + s
    base = wid * GW_BLK
    count = jnp.minimum(GW_BLK, NGRP - base)
    rslice = pl.ds(s * ROWS_PER_TILE, ROWS_PER_TILE)
    pltpu.sync_copy(rowp_hbm.at[pl.ds(base, GW_BLK)], idxs_v)

    def phase(data_hbm, out_hbm):
        pltpu.sync_copy(zro_hbm.at[rslice], acc.at[rslice])
        plsc.subcore_barrier()

        def body(j, carry):
            g0 = (base + 2 * j) * 128
            c0 = pltpu.async_copy(data_hbm.at[pl.ds(g0, 128)], st0_v, sem_a)
            c1 = pltpu.async_copy(data_hbm.at[pl.ds(g0 + 128, 128)], st1_v, sem_b)
            c0.wait()
            s0 = pltpu.async_copy(st0_v, acc.at[idxs_v.at[2 * j]], sem_a, add=True)
            c1.wait()
            s1 = pltpu.async_copy(st1_v, acc.at[idxs_v.at[2 * j + 1]], sem_b, add=True)
            s0.wait()
            s1.wait()
            return carry

        lax.fori_loop(0, count // 2, body, 0)
        plsc.subcore_barrier()
        pltpu.sync_copy(acc.at[rslice], out_hbm.at[c, rslice])
        plsc.subcore_barrier()

    phase(msg_hbm, num_out)
    phase(exb_hbm, den_out)


def _finish_body(v_ref, num_ref, den_ref, wout_ref, o_ref):
    v = v_ref[...]
    num = num_ref[0] + num_ref[1]
    den = den_ref[0] + den_ref[1]
    out = v - num / (den + 1e-9)
    o_ref[...] = jnp.dot(out, wout_ref[...].T, preferred_element_type=jnp.float32)


def kernel(x, edges, edge_index, Wq, Wk, Wv, Wek, Wev, Wexp, Wout):
    scale = HEAD_DIM ** -0.5
    BN = 1000
    q, kv, vfull = pl.pallas_call(
        functools.partial(_proj_body, scale=scale),
        grid=(N // BN,),
        in_specs=[
            pl.BlockSpec((BN, DIM), lambda i: (i, 0)),
            pl.BlockSpec((DIM, DIM), lambda i: (0, 0)),
            pl.BlockSpec((DIM, DIM), lambda i: (0, 0)),
            pl.BlockSpec((DIM, DIM), lambda i: (0, 0)),
        ],
        out_specs=[
            pl.BlockSpec((BN, DIM), lambda i: (i, 0)),
            pl.BlockSpec((BN, DIM), lambda i: (i, 0)),
            pl.BlockSpec((BN, DIM), lambda i: (i, 0)),
        ],
        out_shape=[
            jax.ShapeDtypeStruct((N, DIM), jnp.float32),
            jax.ShapeDtypeStruct((N, DIM), jnp.int32),
            jax.ShapeDtypeStruct((N, DIM), jnp.float32),
        ],
    )(x, Wq, Wk, Wv)

    row = edge_index[0]
    col = edge_index[1]
    zpad = jnp.zeros((NGRP_PAD - NGRP, 128), jnp.int32)
    rowp = jnp.concatenate([row.reshape(NGRP, 128), zpad])
    colp = jnp.concatenate([col.reshape(NGRP, 128), zpad])
    qe, kve = pl.kernel(
        _gather_body,
        out_type=[
            jax.ShapeDtypeStruct((E, DIM), jnp.float32),
            jax.ShapeDtypeStruct((E, DIM), jnp.int32),
        ],
        mesh=plsc.VectorSubcoreMesh(core_axis_name="c", subcore_axis_name="s"),
        scratch_types=[
            pltpu.VMEM((GW_BLK, 128), jnp.int32),
            pltpu.VMEM((GW_BLK, 128), jnp.int32),
            pltpu.VMEM((128, DIM), jnp.float32),
            pltpu.VMEM((128, DIM), jnp.int32),
            pltpu.VMEM((128, DIM), jnp.float32),
            pltpu.VMEM((128, DIM), jnp.int32),
            pltpu.SemaphoreType.DMA,
            pltpu.SemaphoreType.DMA,
            pltpu.SemaphoreType.DMA,
            pltpu.SemaphoreType.DMA,
        ],
    )(q, kv, rowp, colp)

    P = jnp.repeat(Wexp.T, HEAD_DIM, axis=0)
    R = jnp.repeat(jnp.eye(EXP_HEADS, dtype=jnp.float32), HEAD_DIM, axis=1)

    BE = 2000
    msg, exb = pl.pallas_call(
        functools.partial(_edge_body, scale=scale),
        grid=(E // BE,),
        in_specs=[
            pl.BlockSpec((BE, DIM), lambda i: (i, 0)),
            pl.BlockSpec((BE, DIM), lambda i: (i, 0)),
            pl.BlockSpec((BE, DIM), lambda i: (i, 0)),
            pl.BlockSpec((DIM, DIM), lambda i: (0, 0)),
            pl.BlockSpec((DIM, DIM), lambda i: (0, 0)),
            pl.BlockSpec((DIM, EXP_HEADS), lambda i: (0, 0)),
            pl.BlockSpec((EXP_HEADS, DIM), lambda i: (0, 0)),
        ],
        out_specs=[
            pl.BlockSpec((BE, DIM), lambda i: (i, 0)),
            pl.BlockSpec((BE, DIM), lambda i: (i, 0)),
        ],
        out_shape=[
            jax.ShapeDtypeStruct((E, DIM), jnp.float32),
            jax.ShapeDtypeStruct((E, DIM), jnp.float32),
        ],
    )(edges, qe, kve, Wek, Wev, P, R)

    zros = jnp.zeros((NP, DIM), jnp.float32)
    num2, den2 = pl.kernel(
        _scatter_body,
        out_type=[
            jax.ShapeDtypeStruct((2, NP, DIM), jnp.float32),
            jax.ShapeDtypeStruct((2, NP, DIM), jnp.float32),
        ],
        mesh=plsc.VectorSubcoreMesh(core_axis_name="c", subcore_axis_name="s"),
        scratch_types=[
            pltpu.VMEM((GW_BLK, 128), jnp.int32),
            pltpu.VMEM((128, DIM), jnp.float32),
            pltpu.VMEM((128, DIM), jnp.float32),
            pltpu.VMEM_SHARED((NP, DIM), jnp.float32),
            pltpu.SemaphoreType.DMA,
            pltpu.SemaphoreType.DMA,
        ],
    )(msg, exb, rowp, zros)

    out = pl.pallas_call(
        _finish_body,
        grid=(N // BN,),
        in_specs=[
            pl.BlockSpec((BN, DIM), lambda i: (i, 0)),
            pl.BlockSpec((2, BN, DIM), lambda i: (0, i, 0)),
            pl.BlockSpec((2, BN, DIM), lambda i: (0, i, 0)),
            pl.BlockSpec((DIM, DIM), lambda i: (0, 0)),
        ],
        out_specs=pl.BlockSpec((BN, DIM), lambda i: (i, 0)),
        out_shape=jax.ShapeDtypeStruct((N, DIM), jnp.float32),
    )(vfull, num2, den2, Wout)
    return out
